# Initial kernel scaffold; baseline (speedup 1.0000x reference)
import jax
import jax.numpy as jnp
from jax.experimental import pallas as pl


def kernel(x, emb_table, pos_table):
    raise NotImplementedError("write your pallas kernel here")


# speedup vs baseline: 2.0477x; 2.0477x over previous
import functools

import jax
import jax.numpy as jnp
from jax import lax
from jax.experimental import pallas as pl
from jax.experimental.pallas import tpu as pltpu
from jax.experimental.pallas import tpu_sc as plsc

H = 64
CHUNK = 128
LANES = 16


@functools.partial(jax.jit, static_argnums=(3, 4))
def _emb_lookup(x_flat, emb_table, pos_flat, n_rows, seq):
    info = plsc.get_sparse_core_info()
    nw = info.num_cores * info.num_subcores
    rows_per_w = n_rows // nw
    n_chunks = rows_per_w // CHUNK
    posf = seq * H

    mesh = plsc.VectorSubcoreMesh(core_axis_name="c", subcore_axis_name="s")

    @functools.partial(
        pl.kernel,
        mesh=mesh,
        compiler_params=pltpu.CompilerParams(use_tc_tiling_on_sc=False),
        out_type=jax.ShapeDtypeStruct((n_rows, H), jnp.float32),
        scratch_types=[
            pltpu.VMEM((CHUNK,), jnp.int32),
            pltpu.VMEM((CHUNK, H), jnp.float32),
            pltpu.VMEM((2 * posf,), jnp.float32),
            pltpu.SemaphoreType.DMA,
        ],
    )
    def body(x_hbm, emb_hbm, pos_hbm, out_hbm, idx_v, rows_v, pos_v, gsem):
        num_cores = info.num_cores
        wid = lax.axis_index("s") * num_cores + lax.axis_index("c")
        base_w = wid * rows_per_w

        pltpu.sync_copy(pos_hbm, pos_v.at[pl.ds(0, posf)])
        pltpu.sync_copy(pos_hbm, pos_v.at[pl.ds(posf, posf)])

        def do_chunk(c, carry):
            base = base_w + c * CHUNK
            pltpu.sync_copy(x_hbm.at[pl.ds(base, CHUNK)], idx_v)
            pltpu.async_copy(emb_hbm.at[idx_v], rows_v, gsem).wait()

            q = lax.rem(base, seq) * H

            def add_row(r, carry2):
                p = q + r * H
                for j in range(H // LANES):
                    rows_v[r, pl.ds(j * LANES, LANES)] = (
                        rows_v[r, pl.ds(j * LANES, LANES)]
                        + pos_v[pl.ds(p + j * LANES, LANES)]
                    )
                return carry2

            lax.fori_loop(0, CHUNK, add_row, 0)
            pltpu.sync_copy(rows_v, out_hbm.at[pl.ds(base, CHUNK)])
            return carry

        lax.fori_loop(0, n_chunks, do_chunk, 0)

    return body(x_flat, emb_table, pos_flat)


def kernel(x, emb_table, pos_table):
    b, s = x.shape
    n_rows = b * s
    x_flat = x.reshape(n_rows).astype(jnp.int32)
    pos_flat = pos_table.reshape(-1)
    out = _emb_lookup(x_flat, emb_table, pos_flat, n_rows, s)
    return out.reshape(b, s, H)


# speedup vs baseline: 2.7115x; 1.3242x over previous
import functools

import jax
import jax.numpy as jnp
from jax import lax
from jax.experimental import pallas as pl
from jax.experimental.pallas import tpu as pltpu
from jax.experimental.pallas import tpu_sc as plsc

H = 64
CHUNK = 128
LANES = 16
NBUF = 4


@functools.partial(jax.jit, static_argnums=(3, 4))
def _emb_lookup(x_2d, emb_table, pos_flat, n_rows, seq):
    info = plsc.get_sparse_core_info()
    nw = info.num_cores * info.num_subcores
    rows_per_w = n_rows // nw
    n_chunks = rows_per_w // CHUNK
    n_groups = n_chunks // NBUF
    posf = seq * H

    mesh = plsc.VectorSubcoreMesh(core_axis_name="c", subcore_axis_name="s")

    @functools.partial(
        pl.kernel,
        mesh=mesh,
        compiler_params=pltpu.CompilerParams(use_tc_tiling_on_sc=False),
        out_type=jax.ShapeDtypeStruct((n_rows, H), jnp.float32),
        scratch_types=[
            pltpu.VMEM((n_chunks, CHUNK), jnp.int32),
            pltpu.VMEM((NBUF, CHUNK, H), jnp.float32),
            pltpu.VMEM((NBUF, CHUNK, H), jnp.float32),
            pltpu.VMEM((2 * posf,), jnp.float32),
            pltpu.SemaphoreType.DMA((NBUF,)),
            pltpu.SemaphoreType.DMA((NBUF,)),
        ],
    )
    def body(x_hbm, emb_hbm, pos_hbm, out_hbm, idx_v, rows_v, obuf_v,
             pos_v, gsem, osem):
        num_cores = info.num_cores
        wid = lax.axis_index("s") * num_cores + lax.axis_index("c")
        base_w = wid * rows_per_w

        pltpu.sync_copy(x_hbm.at[pl.ds(wid * n_chunks, n_chunks)], idx_v)
        pltpu.sync_copy(pos_hbm, pos_v.at[pl.ds(0, posf)])
        pltpu.sync_copy(pos_hbm, pos_v.at[pl.ds(posf, posf)])

        def start_gather(c, b):
            return pltpu.async_copy(
                emb_hbm.at[idx_v.at[c]], rows_v.at[b], gsem.at[b])

        for b in range(NBUF):
            start_gather(b, b)

        def do_group(g, carry):
            for b in range(NBUF):
                c = g * NBUF + b
                base = base_w + c * CHUNK
                q = lax.rem(c * CHUNK, seq) * H

                pltpu.make_async_copy(
                    emb_hbm.at[idx_v.at[c]], rows_v.at[b], gsem.at[b]
                ).wait()

                @pl.when(g > 0)
                def _wait_prev_out():
                    pltpu.make_async_copy(
                        obuf_v.at[b],
                        out_hbm.at[pl.ds(base - NBUF * CHUNK, CHUNK)],
                        osem.at[b],
                    ).wait()

                def add_row(r, carry2):
                    p = q + r * H
                    for j in range(H // LANES):
                        obuf_v[b, r, pl.ds(j * LANES, LANES)] = (
                            rows_v[b, r, pl.ds(j * LANES, LANES)]
                            + pos_v[pl.ds(p + j * LANES, LANES)]
                        )
                    return carry2

                lax.fori_loop(0, CHUNK, add_row, 0)

                pltpu.async_copy(
                    obuf_v.at[b], out_hbm.at[pl.ds(base, CHUNK)], osem.at[b])

                @pl.when(g < n_groups - 1)
                def _next_gather():
                    start_gather(c + NBUF, b)

            return carry

        lax.fori_loop(0, n_groups, do_group, 0)

        for b in range(NBUF):
            c = n_chunks - NBUF + b
            base = base_w + c * CHUNK
            pltpu.make_async_copy(
                obuf_v.at[b], out_hbm.at[pl.ds(base, CHUNK)], osem.at[b]
            ).wait()

    return body(x_2d, emb_table, pos_flat)


def kernel(x, emb_table, pos_table):
    b, s = x.shape
    n_rows = b * s
    x_2d = x.reshape(n_rows // CHUNK, CHUNK).astype(jnp.int32)
    pos_flat = pos_table.reshape(-1)
    out = _emb_lookup(x_2d, emb_table, pos_flat, n_rows, s)
    return out.reshape(b, s, H)


# speedup vs baseline: 4.2092x; 1.5524x over previous
import functools

import jax
import jax.numpy as jnp
from jax import lax
from jax.experimental import pallas as pl
from jax.experimental.pallas import tpu as pltpu
from jax.experimental.pallas import tpu_sc as plsc

H = 64
CHUNK = 100
LANES = 16
NBUF = 8
AHEAD = 4


@functools.partial(jax.jit, static_argnums=(3, 4))
def _emb_lookup(x_2d, emb_table, pos_table, n_batch, seq):
    info = plsc.get_sparse_core_info()
    nw = info.num_cores * info.num_subcores
    seq_per_w = n_batch // nw
    cpseq = seq // CHUNK
    n_chunks = seq_per_w * cpseq
    n_groups = n_chunks // NBUF

    mesh = plsc.VectorSubcoreMesh(core_axis_name="c", subcore_axis_name="s")

    @functools.partial(
        pl.kernel,
        mesh=mesh,
        compiler_params=pltpu.CompilerParams(use_tc_tiling_on_sc=False),
        out_type=jax.ShapeDtypeStruct((n_batch, seq, H), jnp.float32),
        scratch_types=[
            pltpu.VMEM((n_chunks, CHUNK), jnp.int32),
            pltpu.VMEM((NBUF, CHUNK, H), jnp.float32),
            pltpu.VMEM((seq, H), jnp.float32),
            pltpu.SemaphoreType.DMA((NBUF,)),
            pltpu.SemaphoreType.DMA((NBUF,)),
        ],
    )
    def body(x_hbm, emb_hbm, pos_hbm, out_hbm, idx_v, rows_v, pos_v,
             gsem, osem):
        num_cores = info.num_cores
        wid = lax.axis_index("s") * num_cores + lax.axis_index("c")
        seq0 = wid * seq_per_w

        pltpu.sync_copy(x_hbm.at[pl.ds(wid * n_chunks, n_chunks)], idx_v)
        pltpu.sync_copy(pos_hbm, pos_v)

        def out_slice(c, b):
            s_idx = seq0 + c // cpseq
            phase = (b % cpseq) * CHUNK
            return out_hbm.at[s_idx, pl.ds(phase, CHUNK), :]

        def start_gather(c, b):
            pltpu.async_copy(emb_hbm.at[idx_v.at[c]], rows_v.at[b],
                             gsem.at[b])

        for d in range(AHEAD):
            start_gather(d, d)

        def do_group(g, carry):
            for b in range(NBUF):
                c = g * NBUF + b
                phase = (b % cpseq) * CHUNK

                pltpu.make_async_copy(
                    emb_hbm.at[idx_v.at[c]], rows_v.at[b], gsem.at[b]
                ).wait()

                @plsc.parallel_loop(0, CHUNK, step=1, unroll=4)
                def add_row(r):
                    for j in range(H // LANES):
                        plsc.addupdate(
                            rows_v.at[b, r, pl.ds(j * LANES, LANES)],
                            pos_v[phase + r, pl.ds(j * LANES, LANES)],
                        )

                pltpu.async_copy(rows_v.at[b], out_slice(c, b), osem.at[b])

                bn = (b + AHEAD) % NBUF

                @pl.when(c + AHEAD < n_chunks)
                def _prefetch():
                    @pl.when(c >= NBUF - AHEAD)
                    def _wait_prev_out():
                        cp = c + AHEAD - NBUF
                        pltpu.make_async_copy(
                            rows_v.at[bn], out_slice(cp, bn), osem.at[bn]
                        ).wait()

                    start_gather(c + AHEAD, bn)

            return carry

        lax.fori_loop(0, n_groups, do_group, 0)

        for b in range(NBUF):
            c = n_chunks - NBUF + b
            pltpu.make_async_copy(
                rows_v.at[b], out_slice(c, b), osem.at[b]
            ).wait()

    return body(x_2d, emb_table, pos_table)


def kernel(x, emb_table, pos_table):
    b, s = x.shape
    x_2d = x.reshape(b * s // CHUNK, CHUNK).astype(jnp.int32)
    return _emb_lookup(x_2d, emb_table, pos_table, b, s)


# speedup vs baseline: 5.5420x; 1.3166x over previous
import functools

import jax
import jax.numpy as jnp
from jax import lax
from jax.experimental import pallas as pl
from jax.experimental.pallas import tpu as pltpu
from jax.experimental.pallas import tpu_sc as plsc

H = 64
HP = 128
CHUNK = 128
LANES = 16
NBUF = 3
AHEAD = 2
IDXN = 4


@functools.partial(jax.jit, static_argnums=(3, 4))
def _emb_lookup(x_flat, emb_pad, pos_table, n_rows, seq):
    info = plsc.get_sparse_core_info()
    nw = info.num_cores * info.num_subcores
    rows_per_w = n_rows // nw
    n_chunks = rows_per_w // CHUNK

    mesh = plsc.VectorSubcoreMesh(core_axis_name="c", subcore_axis_name="s")

    @functools.partial(
        pl.kernel,
        mesh=mesh,
        out_type=jax.ShapeDtypeStruct((n_rows, H), jnp.float32),
        scratch_types=[
            pltpu.VMEM((IDXN * CHUNK,), jnp.int32),
            pltpu.VMEM((NBUF, CHUNK, HP), jnp.float32),
            pltpu.VMEM((NBUF, CHUNK, H), jnp.float32),
            pltpu.VMEM((seq, H), jnp.float32),
            pltpu.SemaphoreType.DMA((NBUF,)),
            pltpu.SemaphoreType.DMA((NBUF,)),
            pltpu.SemaphoreType.DMA((IDXN,)),
        ],
    )
    def body(x_hbm, emb_hbm, pos_hbm, out_hbm, idx_v, g_v, rows_v, pos_v,
             gsem, osem, isem):
        num_cores = info.num_cores
        wid = lax.axis_index("s") * num_cores + lax.axis_index("c")
        row0 = wid * rows_per_w

        pltpu.sync_copy(pos_hbm, pos_v)

        def idx_copy(c):
            k = lax.rem(c, IDXN)
            return pltpu.make_async_copy(
                x_hbm.at[pl.ds(row0 + c * CHUNK, CHUNK)],
                idx_v.at[pl.ds(pl.multiple_of(k * CHUNK, CHUNK), CHUNK)],
                isem.at[k])

        def gather_copy(c):
            k = lax.rem(c, IDXN)
            b = lax.rem(c, NBUF)
            return pltpu.make_async_copy(
                emb_hbm.at[idx_v.at[pl.ds(pl.multiple_of(k * CHUNK, CHUNK),
                                          CHUNK)]],
                g_v.at[b], gsem.at[b])

        def out_copy(c):
            b = lax.rem(c, NBUF)
            return pltpu.make_async_copy(
                rows_v.at[b],
                out_hbm.at[pl.ds(row0 + c * CHUNK, CHUNK), :],
                osem.at[b])

        for k in range(IDXN):
            idx_copy(k).start()
        for d in range(AHEAD):
            idx_copy(d).wait()
            gather_copy(d).start()

        def do_chunk(c, carry):
            b = lax.rem(c, NBUF)
            gather_copy(c).wait()

            @pl.when(c >= NBUF)
            def _wait_prev_out():
                out_copy(c - NBUF).wait()

            q = lax.rem(c * CHUNK, seq)

            @plsc.parallel_loop(0, CHUNK, step=1, unroll=4)
            def add_row(r):
                s_pos = lax.rem(q + r, seq)
                for j in range(H // LANES):
                    rows_v[b, r, pl.ds(j * LANES, LANES)] = (
                        g_v[b, r, pl.ds(j * LANES, LANES)]
                        + pos_v[s_pos, pl.ds(j * LANES, LANES)]
                    )

            out_copy(c).start()

            @pl.when(c + IDXN < n_chunks)
            def _idx_prefetch():
                idx_copy(c + IDXN).start()

            @pl.when(c + AHEAD < n_chunks)
            def _gather_prefetch():
                idx_copy(c + AHEAD).wait()
                gather_copy(c + AHEAD).start()

            return carry

        lax.fori_loop(0, n_chunks, do_chunk, 0)

        for c in range(n_chunks - NBUF, n_chunks):
            out_copy(c).wait()

    return body(x_flat, emb_pad, pos_table)


def kernel(x, emb_table, pos_table):
    b, s = x.shape
    x_flat = x.reshape(-1).astype(jnp.int32)
    emb_pad = jnp.pad(emb_table, ((0, 0), (0, HP - H)))
    out = _emb_lookup(x_flat, emb_pad, pos_table, b * s, s)
    return out.reshape(b, s, H)
